# baseline (speedup 1.0000x reference)
import functools

import jax
import jax.numpy as jnp
from jax import lax
from jax.experimental import pallas as pl
from jax.experimental.pallas import tpu as pltpu
from jax.experimental.pallas import tpu_sc as plsc

N_FACTORS = 64
BATCH = 16384

_info = plsc.get_sparse_core_info()
NC = _info.num_cores
NS = _info.num_subcores
LANES = _info.num_lanes
NW = NC * NS
BPW = BATCH // NW
GCHUNK = 128
NCHUNK = BPW // GCHUNK


def _body(user_hbm, book_hbm, uemb_hbm, bemb_hbm, out_hbm,
          uidx_v, bidx_v, urows_v, brows_v, out_v, sem_u, sem_b):
  wid = lax.axis_index("s") * NC + lax.axis_index("c")
  base = wid * BPW

  pltpu.sync_copy(user_hbm.at[pl.ds(base, BPW)], uidx_v)
  pltpu.sync_copy(book_hbm.at[pl.ds(base, BPW)], bidx_v)

  copies = []
  for k in range(NCHUNK):
    sl = pl.ds(k * GCHUNK, GCHUNK)
    copies.append(pltpu.async_copy(
        uemb_hbm.at[uidx_v.at[sl]], urows_v.at[sl], sem_u))
    copies.append(pltpu.async_copy(
        bemb_hbm.at[bidx_v.at[sl]], brows_v.at[sl], sem_b))
  for c in copies:
    c.wait()

  lane = lax.iota(jnp.int32, LANES)
  one = jnp.ones((LANES,), jnp.int32)

  def group(g, carry):
    rows = g * LANES + lane
    col = jnp.zeros((LANES,), jnp.int32)
    acc = jnp.zeros((LANES,), jnp.float32)
    for j in range(N_FACTORS):
      u = plsc.load_gather(urows_v, [rows, col])
      b = plsc.load_gather(brows_v, [rows, col])
      acc = acc + u * b
      if j + 1 < N_FACTORS:
        col = col + one
    out_v[pl.ds(g * LANES, LANES)] = acc
    return carry

  lax.fori_loop(0, BPW // LANES, group, 0)

  pltpu.sync_copy(out_v, out_hbm.at[pl.ds(base, BPW)])


@jax.jit
def kernel(user, book, user_emb, book_emb):
  mesh = plsc.VectorSubcoreMesh(core_axis_name="c", subcore_axis_name="s")
  run = functools.partial(
      pl.kernel,
      out_type=jax.ShapeDtypeStruct((BATCH,), jnp.float32),
      mesh=mesh,
      compiler_params=pltpu.CompilerParams(
          use_tc_tiling_on_sc=False, needs_layout_passes=False),
      scratch_types=[
          pltpu.VMEM((BPW,), jnp.int32),
          pltpu.VMEM((BPW,), jnp.int32),
          pltpu.VMEM((BPW, N_FACTORS), jnp.float32),
          pltpu.VMEM((BPW, N_FACTORS), jnp.float32),
          pltpu.VMEM((BPW,), jnp.float32),
          pltpu.SemaphoreType.DMA,
          pltpu.SemaphoreType.DMA,
      ],
  )(_body)
  return run(user.astype(jnp.int32), book.astype(jnp.int32),
             user_emb, book_emb)
